# Initial kernel scaffold; baseline (speedup 1.0000x reference)
import jax
import jax.numpy as jnp
from jax.experimental import pallas as pl


def kernel(video_feat, tag_feat, tag_embed, gamma_v, beta_v, gamma_t, beta_t, W_video, b_video, W_tag, b_tag, W_vv_0, W_tv_0, W_vt_0, b_v_0, b_t_0, W_vv_1, W_tv_1, W_vt_1, b_v_1, b_t_1, tag_nids, edge_index_vv, src_tv, dst_tv, src_vt, dst_vt):
    raise NotImplementedError("write your pallas kernel here")


# speedup vs baseline: 1.0908x; 1.0908x over previous
import functools

import jax
import jax.numpy as jnp
from jax import lax
from jax.experimental import pallas as pl
from jax.experimental.pallas import tpu as pltpu
from jax.experimental.pallas import tpu_sc as plsc

_NC, _NS = 2, 16

N_VIDEO = 50000
N_TAG = 5000
NUM_TAGS = 100000
VIDEO_IN = 128
TAG_IN = 64
HID = 128

_ROWS = 1000


def _stats1_body(x_ref, o_ref):
    @pl.when(pl.program_id(0) == 0)
    def _():
        o_ref[...] = jnp.zeros_like(o_ref)

    x = x_ref[...]
    s1 = jnp.sum(x, axis=0, keepdims=True)
    s2 = jnp.sum(x * x, axis=0, keepdims=True)
    o_ref[...] += jnp.concatenate([s1, s2], axis=0)


def _stats2_body(x_ref, y_ref, o_ref):
    @pl.when(pl.program_id(0) == 0)
    def _():
        o_ref[...] = jnp.zeros_like(o_ref)

    x = x_ref[...] + y_ref[...]
    s1 = jnp.sum(x, axis=0, keepdims=True)
    s2 = jnp.sum(x * x, axis=0, keepdims=True)
    o_ref[...] += jnp.concatenate([s1, s2], axis=0)


def _col_stats(x, y=None):
    n, d = x.shape
    grid = n // _ROWS
    xspec = pl.BlockSpec((_ROWS, d), lambda i: (i, 0))
    ospec = pl.BlockSpec((2, d), lambda i: (0, 0))
    if y is None:
        return pl.pallas_call(
            _stats1_body, grid=(grid,), in_specs=[xspec],
            out_specs=ospec,
            out_shape=jax.ShapeDtypeStruct((2, d), jnp.float32),
        )(x)
    return pl.pallas_call(
        _stats2_body, grid=(grid,), in_specs=[xspec, xspec],
        out_specs=ospec,
        out_shape=jax.ShapeDtypeStruct((2, d), jnp.float32),
    )(x, y)


def _enc1_body(x_ref, sc_ref, sh_ref, w_ref, b_ref, o_ref):
    xn = x_ref[...] * sc_ref[...] + sh_ref[...]
    o_ref[...] = jax.nn.gelu(
        jnp.dot(xn, w_ref[...], preferred_element_type=jnp.float32) + b_ref[...])


def _enc2_body(x_ref, y_ref, sc_ref, sh_ref, w_ref, b_ref, o_ref):
    xn = (x_ref[...] + y_ref[...]) * sc_ref[...] + sh_ref[...]
    o_ref[...] = jax.nn.gelu(
        jnp.dot(xn, w_ref[...], preferred_element_type=jnp.float32) + b_ref[...])


def _encode(x, scale, shift, w, b, y=None):
    n, d = x.shape
    h = w.shape[1]
    grid = n // _ROWS
    xspec = pl.BlockSpec((_ROWS, d), lambda i: (i, 0))
    vspec = pl.BlockSpec((1, d), lambda i: (0, 0))
    wspec = pl.BlockSpec((d, h), lambda i: (0, 0))
    bspec = pl.BlockSpec((1, h), lambda i: (0, 0))
    ospec = pl.BlockSpec((_ROWS, h), lambda i: (i, 0))
    oshape = jax.ShapeDtypeStruct((n, h), jnp.float32)
    args = (x, scale.reshape(1, d), shift.reshape(1, d), w, b.reshape(1, h))
    if y is None:
        return pl.pallas_call(
            _enc1_body, grid=(grid,),
            in_specs=[xspec, vspec, vspec, wspec, bspec],
            out_specs=ospec, out_shape=oshape)(*args)
    return pl.pallas_call(
        _enc2_body, grid=(grid,),
        in_specs=[xspec, xspec, vspec, vspec, wspec, bspec],
        out_specs=ospec, out_shape=oshape)(x, y, *args[1:])


def _updv_body(sv_ref, st_ref, cv_ref, ct_ref, h_ref, wv_ref, wt_ref, b_ref,
               o_ref, *, final):
    aggv = sv_ref[...] / jnp.maximum(cv_ref[...][:, :1], 1.0)
    aggt = st_ref[...] / jnp.maximum(ct_ref[...][:, :1], 1.0)
    u = (jnp.dot(aggv, wv_ref[...], preferred_element_type=jnp.float32)
         + jnp.dot(aggt, wt_ref[...], preferred_element_type=jnp.float32)
         + b_ref[...])
    h = h_ref[...] + jax.nn.gelu(u)
    if final:
        nrm = jnp.sqrt(jnp.sum(h * h, axis=1, keepdims=True))
        h = h / jnp.maximum(nrm, 1e-12)
    o_ref[...] = h


def _update_v(sum_vv, sum_tv, cnt_vv, cnt_tv, h_v, wvv, wtv, bv, final):
    n, d = h_v.shape
    grid = n // _ROWS
    mspec = pl.BlockSpec((_ROWS, d), lambda i: (i, 0))
    cspec = pl.BlockSpec((_ROWS, 16), lambda i: (i, 0))
    wspec = pl.BlockSpec((d, d), lambda i: (0, 0))
    bspec = pl.BlockSpec((1, d), lambda i: (0, 0))
    return pl.pallas_call(
        functools.partial(_updv_body, final=final), grid=(grid,),
        in_specs=[mspec, mspec, cspec, cspec, mspec, wspec, wspec, bspec],
        out_specs=mspec,
        out_shape=jax.ShapeDtypeStruct((n, d), jnp.float32),
    )(sum_vv, sum_tv, cnt_vv, cnt_tv, h_v, wvv, wtv, bv.reshape(1, d))


def _updt_body(sv_ref, cv_ref, h_ref, w_ref, b_ref, o_ref):
    agg = sv_ref[...] / jnp.maximum(cv_ref[...][:, :1], 1.0)
    u = jnp.dot(agg, w_ref[...], preferred_element_type=jnp.float32) + b_ref[...]
    o_ref[...] = h_ref[...] + jax.nn.gelu(u)


def _update_t(sum_vt, cnt_vt, h_t, wvt, bt):
    n, d = h_t.shape
    grid = n // _ROWS
    mspec = pl.BlockSpec((_ROWS, d), lambda i: (i, 0))
    cspec = pl.BlockSpec((_ROWS, 16), lambda i: (i, 0))
    wspec = pl.BlockSpec((d, d), lambda i: (0, 0))
    bspec = pl.BlockSpec((1, d), lambda i: (0, 0))
    return pl.pallas_call(
        _updt_body, grid=(grid,),
        in_specs=[mspec, cspec, mspec, wspec, bspec],
        out_specs=mspec,
        out_shape=jax.ShapeDtypeStruct((n, d), jnp.float32),
    )(sum_vt, cnt_vt, h_t, wvt, bt.reshape(1, d))


def _pad_to(x, n, value):
    return jnp.concatenate(
        [x.astype(jnp.int32), jnp.full((n - x.shape[0],), value, jnp.int32)])


def _seg_sum(h_src, src, dst, n_dst):
    msg = jnp.take(h_src, src, axis=0)
    return jax.ops.segment_sum(msg, dst, num_segments=n_dst)


def _counts(dst, n_dst):
    ones = jnp.ones((dst.shape[0], 16), jnp.float32)
    return jax.ops.segment_sum(ones, dst, num_segments=n_dst)


@functools.lru_cache(maxsize=None)
def _make_tag_gather(n_pad, d):
    per_w = n_pad // (_NC * _NS)
    mesh = plsc.VectorSubcoreMesh(core_axis_name="c", subcore_axis_name="s")

    def body(table_hbm, nids_hbm, out_hbm, idx_v, rows_v, sem):
        wid = lax.axis_index("s") * _NC + lax.axis_index("c")
        base = wid * per_w
        for p in range(per_w // 80):
            off = base + p * 80
            pltpu.sync_copy(nids_hbm.at[pl.ds(off, 80)], idx_v)
            pltpu.async_copy(table_hbm.at[idx_v], rows_v, sem).wait()
            pltpu.sync_copy(rows_v, out_hbm.at[pl.ds(off, 80)])

    return pl.kernel(
        body, mesh=mesh,
        out_type=jax.ShapeDtypeStruct((n_pad, d), jnp.float32),
        compiler_params=pltpu.CompilerParams(use_tc_tiling_on_sc=False),
        scratch_types=[pltpu.VMEM((80,), jnp.int32),
                       pltpu.VMEM((80, d), jnp.float32),
                       pltpu.SemaphoreType.DMA])


def _tag_gather(tag_embed, tag_nids):
    n_pad = 5120
    nids = _pad_to(tag_nids, n_pad, 0)
    return _make_tag_gather(n_pad, tag_embed.shape[1])(tag_embed, nids)


def _bn_scale_shift(stats, n, gamma, beta):
    mu = stats[0] / n
    var = stats[1] / n - mu * mu
    scale = gamma * jax.lax.rsqrt(var + 1e-5)
    return scale, beta - mu * scale


def kernel(video_feat, tag_feat, tag_embed, gamma_v, beta_v, gamma_t, beta_t,
           W_video, b_video, W_tag, b_tag,
           W_vv_0, W_tv_0, W_vt_0, b_v_0, b_t_0,
           W_vv_1, W_tv_1, W_vt_1, b_v_1, b_t_1,
           tag_nids, edge_index_vv, src_tv, dst_tv, src_vt, dst_vt):
    src_vv, dst_vv = edge_index_vv[0], edge_index_vv[1]

    tag_rows = _tag_gather(tag_embed, tag_nids)

    stats_v = _col_stats(video_feat)
    stats_t = _col_stats(tag_feat, tag_rows)
    scale_v, shift_v = _bn_scale_shift(stats_v, N_VIDEO, gamma_v, beta_v)
    scale_t, shift_t = _bn_scale_shift(stats_t, N_TAG, gamma_t, beta_t)

    h_v = _encode(video_feat, scale_v, shift_v, W_video, b_video)
    h_t = _encode(tag_feat, scale_t, shift_t, W_tag, b_tag, y=tag_rows)

    cnt_vv = _counts(dst_vv, N_VIDEO)
    cnt_tv = _counts(dst_tv, N_VIDEO)
    cnt_vt = _counts(dst_vt, N_TAG)

    sum_vv = _seg_sum(h_v, src_vv, dst_vv, N_VIDEO)
    sum_tv = _seg_sum(h_t, src_tv, dst_tv, N_VIDEO)
    sum_vt = _seg_sum(h_v, src_vt, dst_vt, N_TAG)
    h_v1 = _update_v(sum_vv, sum_tv, cnt_vv, cnt_tv, h_v, W_vv_0, W_tv_0,
                     b_v_0, final=False)
    h_t1 = _update_t(sum_vt, cnt_vt, h_t, W_vt_0, b_t_0)

    sum_vv1 = _seg_sum(h_v1, src_vv, dst_vv, N_VIDEO)
    sum_tv1 = _seg_sum(h_t1, src_tv, dst_tv, N_VIDEO)
    return _update_v(sum_vv1, sum_tv1, cnt_vv, cnt_tv, h_v1, W_vv_1, W_tv_1,
                     b_v_1, final=True)
